# baseline (speedup 1.0000x reference)
import jax
import jax.numpy as jnp
from jax.experimental import pallas as pl

S_BLOCK = 10


def _pe_add_kernel(x_ref, pe_ref, out_ref):
    out_ref[...] = x_ref[...] + pe_ref[...][:, :, None, :]


def kernel(x, pos_embed):
    S, B, D = x.shape
    n = S // S_BLOCK
    x4 = x.reshape(n, S_BLOCK, B, D)
    pe3 = pos_embed[:S].reshape(n, S_BLOCK, D)
    out = pl.pallas_call(
        _pe_add_kernel,
        grid=(n,),
        in_specs=[
            pl.BlockSpec((1, S_BLOCK, B, D), lambda i: (i, 0, 0, 0)),
            pl.BlockSpec((1, S_BLOCK, D), lambda i: (i, 0, 0)),
        ],
        out_specs=pl.BlockSpec((1, S_BLOCK, B, D), lambda i: (i, 0, 0, 0)),
        out_shape=jax.ShapeDtypeStruct((n, S_BLOCK, B, D), x.dtype),
    )(x4, pe3)
    return out.reshape(S, B, D)


# speedup vs baseline: 1.0153x; 1.0153x over previous
import jax
import jax.numpy as jnp
from jax.experimental import pallas as pl

S_BLOCK = 25


def _pe_add_kernel(x_ref, pe_ref, out_ref):
    out_ref[...] = x_ref[...] + pe_ref[...][:, :, None, :]


def kernel(x, pos_embed):
    S, B, D = x.shape
    n = S // S_BLOCK
    x4 = x.reshape(n, S_BLOCK, B, D)
    pe3 = pos_embed[:S].reshape(n, S_BLOCK, D)
    out = pl.pallas_call(
        _pe_add_kernel,
        grid=(n,),
        in_specs=[
            pl.BlockSpec((1, S_BLOCK, B, D), lambda i: (i, 0, 0, 0)),
            pl.BlockSpec((1, S_BLOCK, D), lambda i: (i, 0, 0)),
        ],
        out_specs=pl.BlockSpec((1, S_BLOCK, B, D), lambda i: (i, 0, 0, 0)),
        out_shape=jax.ShapeDtypeStruct((n, S_BLOCK, B, D), x.dtype),
    )(x4, pe3)
    return out.reshape(S, B, D)
